# baseline (speedup 1.0000x reference)
import functools

import jax
import jax.numpy as jnp
from jax import lax
from jax.experimental import pallas as pl
from jax.experimental.pallas import tpu as pltpu
from jax.experimental.pallas import tpu_sc as plsc

_VMAX_SAFE = 2.0 ** 34


def _huber_of(xt, pt):
    ab = lax.dot_general(xt, pt, (((1,), (0,)), ((), ())),
                         preferred_element_type=jnp.float32)
    a2 = jnp.sum(xt * xt, axis=1, keepdims=True)
    b2 = jnp.sum(pt * pt, axis=0, keepdims=True)
    d2 = a2 + b2 - 2.0 * ab
    dist = jnp.sqrt(jnp.maximum(d2, 0.0)) + 1e-20
    return (1.0 / 1.0) * (jnp.exp(1.0 * dist) + jnp.exp(-1.0 * dist) - (-1.0))


def _argmax_update(nt, n, v, huber_unused, bn, idx_ref, vmax_ref, run_v, run_arg):
    bm = v.shape[0]
    vmax = jnp.max(v, axis=1, keepdims=True)
    col = lax.broadcasted_iota(jnp.int32, v.shape, 1)
    big = jnp.int32(nt * bn)
    arg_loc = jnp.min(jnp.where(v == vmax, col, big), axis=1, keepdims=True)
    garg = n * bn + arg_loc

    @pl.when(n == 0)
    def _():
        run_v[...] = jnp.full((bm, 1), -jnp.inf, jnp.float32)
        run_arg[...] = jnp.zeros((bm, 1), jnp.int32)

    better = vmax > run_v[...]
    new_v = jnp.where(better, vmax, run_v[...])
    new_arg = jnp.where(better, garg, run_arg[...])
    run_v[...] = new_v
    run_arg[...] = new_arg

    @pl.when(n == nt - 1)
    def _():
        idx_ref[...] = jnp.broadcast_to(new_arg, idx_ref.shape)
        if vmax_ref is not None:
            vmax_ref[...] = jnp.broadcast_to(new_v, vmax_ref.shape)


def _select_body(nt, bm, bn, x_ref, p_ref, idx_ref, vmax_ref, run_v, run_arg):
    n = pl.program_id(1)
    v = _huber_of(x_ref[...], p_ref[...])
    _argmax_update(nt, n, v, None, bn, idx_ref, vmax_ref, run_v, run_arg)


def _select_g_body(nt, bm, bn, x_ref, p_ref, g_ref, idx_ref, run_v, run_arg):
    n = pl.program_id(1)
    v = _huber_of(x_ref[...], p_ref[...]) + g_ref[...]
    _argmax_update(nt, n, v, None, bn, idx_ref, None, run_v, run_arg)


def _select_call(xf, protos_t, bm, bn):
    m, k = xf.shape
    n = protos_t.shape[1]
    mt, nt = m // bm, n // bn
    body = functools.partial(_select_body, nt, bm, bn)
    return pl.pallas_call(
        body,
        grid=(mt, nt),
        in_specs=[
            pl.BlockSpec((bm, k), lambda i, j: (i, 0)),
            pl.BlockSpec((k, bn), lambda i, j: (0, j)),
        ],
        out_specs=[
            pl.BlockSpec((bm, 128), lambda i, j: (i, 0)),
            pl.BlockSpec((bm, 128), lambda i, j: (i, 0)),
        ],
        out_shape=[
            jax.ShapeDtypeStruct((m, 128), jnp.int32),
            jax.ShapeDtypeStruct((m, 128), jnp.float32),
        ],
        scratch_shapes=[
            pltpu.VMEM((bm, 1), jnp.float32),
            pltpu.VMEM((bm, 1), jnp.int32),
        ],
    )(xf, protos_t)


def _select_g_call(xf, protos_t, g, bm, bn):
    m, k = xf.shape
    n = protos_t.shape[1]
    mt, nt = m // bm, n // bn
    body = functools.partial(_select_g_body, nt, bm, bn)
    return pl.pallas_call(
        body,
        grid=(mt, nt),
        in_specs=[
            pl.BlockSpec((bm, k), lambda i, j: (i, 0)),
            pl.BlockSpec((k, bn), lambda i, j: (0, j)),
            pl.BlockSpec((bm, bn), lambda i, j: (i, j)),
        ],
        out_specs=pl.BlockSpec((bm, 128), lambda i, j: (i, 0)),
        out_shape=jax.ShapeDtypeStruct((m, 128), jnp.int32),
        scratch_shapes=[
            pltpu.VMEM((bm, 1), jnp.float32),
            pltpu.VMEM((bm, 1), jnp.int32),
        ],
    )(xf, protos_t, g)


_SC_CHUNK = 32


def _gather_body(b_per_w, d, table_hbm, idx_hbm, out_hbm, idx_v, rows_v, sem):
    wid = lax.axis_index("s") * 2 + lax.axis_index("c")
    base = wid * b_per_w
    pltpu.sync_copy(idx_hbm.at[pl.ds(base, b_per_w)], idx_v)
    for c in range(b_per_w // _SC_CHUNK):
        pltpu.async_copy(
            table_hbm.at[idx_v.at[pl.ds(c * _SC_CHUNK, _SC_CHUNK)]],
            rows_v, sem).wait()
        pltpu.sync_copy(rows_v,
                        out_hbm.at[pl.ds(base + c * _SC_CHUNK, _SC_CHUNK)])


def _gather_call(prototypes, idx):
    b = idx.shape[0]
    d = prototypes.shape[1]
    nw = 32
    b_per_w = b // nw
    mesh = plsc.VectorSubcoreMesh(core_axis_name="c", subcore_axis_name="s")
    body = functools.partial(_gather_body, b_per_w, d)
    return pl.kernel(
        body,
        out_type=jax.ShapeDtypeStruct((b, d), jnp.float32),
        mesh=mesh,
        scratch_types=[
            pltpu.VMEM((b_per_w,), jnp.int32),
            pltpu.VMEM((_SC_CHUNK, d), jnp.float32),
            pltpu.SemaphoreType.DMA,
        ],
    )(prototypes, idx)


def _residual_body(x_ref, p_ref, o_ref, h_ref):
    xr = x_ref[...] - p_ref[...]
    o_ref[...] = xr
    d2 = jnp.sum(xr * xr, axis=1, keepdims=True)
    dist = jnp.sqrt(jnp.maximum(d2, 0.0)) + 1e-20
    hsel = (1.0 / 1.0) * (jnp.exp(1.0 * dist) + jnp.exp(-1.0 * dist) - (-1.0))
    h_ref[...] = jnp.broadcast_to(hsel, h_ref.shape)


def _residual_call(xf, proto, bm):
    m, k = xf.shape
    return pl.pallas_call(
        _residual_body,
        grid=(m // bm,),
        in_specs=[
            pl.BlockSpec((bm, k), lambda i: (i, 0)),
            pl.BlockSpec((bm, k), lambda i: (i, 0)),
        ],
        out_specs=[
            pl.BlockSpec((bm, k), lambda i: (i, 0)),
            pl.BlockSpec((bm, 128), lambda i: (i, 0)),
        ],
        out_shape=[
            jax.ShapeDtypeStruct((m, k), jnp.float32),
            jax.ShapeDtypeStruct((m, 128), jnp.float32),
        ],
    )(xf, proto)


_G_CACHE = {}


def _gumbel_table(shape):
    if shape not in _G_CACHE:
        u = jax.random.uniform(jax.random.key(42), shape,
                               minval=1e-20, maxval=1.0)
        _G_CACHE[shape] = -jnp.log(-jnp.log(u))
    return _G_CACHE[shape]


def kernel(x, prototypes):
    batch, seq, hidden = x.shape
    m = batch * seq
    xf = x.reshape(m, hidden)
    pt = prototypes.T

    idx_w, vmax_w = _select_call(xf, pt, bm=1024, bn=1024)
    idx_fast = idx_w[:, 0]
    vmax = vmax_w[:, 0]

    g = _gumbel_table((m, prototypes.shape[0]))
    need_gumbel = jnp.any(vmax < jnp.float32(_VMAX_SAFE))
    idx = lax.cond(
        need_gumbel,
        lambda: _select_g_call(xf, pt, g, bm=1024, bn=1024)[:, 0],
        lambda: idx_fast,
    )

    proto = _gather_call(prototypes, idx)
    xr, hsel_w = _residual_call(xf, proto, bm=512)
    hsel = hsel_w[:, :1]

    return (proto.reshape(batch, seq, hidden),
            xr.reshape(batch, seq, hidden),
            hsel)


# speedup vs baseline: 3.2405x; 3.2405x over previous
import functools

import jax
import jax.numpy as jnp
from jax import lax
from jax.experimental import pallas as pl
from jax.experimental.pallas import tpu as pltpu
from jax.experimental.pallas import tpu_sc as plsc

_VMAX_SAFE = 2.0 ** 34


def _huber_of(xt, pt):
    ab = lax.dot_general(xt, pt, (((1,), (0,)), ((), ())),
                         preferred_element_type=jnp.float32)
    a2 = jnp.sum(xt * xt, axis=1, keepdims=True)
    b2 = jnp.sum(pt * pt, axis=0, keepdims=True)
    d2 = a2 + b2 - 2.0 * ab
    dist = jnp.sqrt(jnp.maximum(d2, 0.0)) + 1e-20
    return (1.0 / 1.0) * (jnp.exp(1.0 * dist) + jnp.exp(-1.0 * dist) - (-1.0))


def _argmax_update(nt, n, v, huber_unused, bn, idx_ref, vmax_ref, run_v, run_arg):
    bm = v.shape[0]
    vmax = jnp.max(v, axis=1, keepdims=True)
    col = lax.broadcasted_iota(jnp.int32, v.shape, 1)
    big = jnp.int32(nt * bn)
    arg_loc = jnp.min(jnp.where(v == vmax, col, big), axis=1, keepdims=True)
    garg = n * bn + arg_loc

    @pl.when(n == 0)
    def _():
        run_v[...] = jnp.full((bm, 1), -jnp.inf, jnp.float32)
        run_arg[...] = jnp.zeros((bm, 1), jnp.int32)

    better = vmax > run_v[...]
    new_v = jnp.where(better, vmax, run_v[...])
    new_arg = jnp.where(better, garg, run_arg[...])
    run_v[...] = new_v
    run_arg[...] = new_arg

    @pl.when(n == nt - 1)
    def _():
        idx_ref[...] = jnp.broadcast_to(new_arg, idx_ref.shape)
        if vmax_ref is not None:
            vmax_ref[...] = jnp.broadcast_to(new_v, vmax_ref.shape)


def _select_body(nt, bm, bn, x_ref, p_ref, idx_ref, vmax_ref, run_v, run_arg):
    n = pl.program_id(1)
    v = _huber_of(x_ref[...], p_ref[...])
    _argmax_update(nt, n, v, None, bn, idx_ref, vmax_ref, run_v, run_arg)


def _select_g_body(nt, bm, bn, x_ref, p_ref, g_ref, idx_ref, run_v, run_arg):
    n = pl.program_id(1)
    v = _huber_of(x_ref[...], p_ref[...]) + g_ref[...]
    _argmax_update(nt, n, v, None, bn, idx_ref, None, run_v, run_arg)


def _select_call(xf, protos_t, bm, bn):
    m, k = xf.shape
    n = protos_t.shape[1]
    mt, nt = m // bm, n // bn
    body = functools.partial(_select_body, nt, bm, bn)
    return pl.pallas_call(
        body,
        grid=(mt, nt),
        in_specs=[
            pl.BlockSpec((bm, k), lambda i, j: (i, 0)),
            pl.BlockSpec((k, bn), lambda i, j: (0, j)),
        ],
        out_specs=[
            pl.BlockSpec((bm, 128), lambda i, j: (i, 0)),
            pl.BlockSpec((bm, 128), lambda i, j: (i, 0)),
        ],
        out_shape=[
            jax.ShapeDtypeStruct((m, 128), jnp.int32),
            jax.ShapeDtypeStruct((m, 128), jnp.float32),
        ],
        scratch_shapes=[
            pltpu.VMEM((bm, 1), jnp.float32),
            pltpu.VMEM((bm, 1), jnp.int32),
        ],
    )(xf, protos_t)


def _select_g_call(xf, protos_t, g, bm, bn):
    m, k = xf.shape
    n = protos_t.shape[1]
    mt, nt = m // bm, n // bn
    body = functools.partial(_select_g_body, nt, bm, bn)
    return pl.pallas_call(
        body,
        grid=(mt, nt),
        in_specs=[
            pl.BlockSpec((bm, k), lambda i, j: (i, 0)),
            pl.BlockSpec((k, bn), lambda i, j: (0, j)),
            pl.BlockSpec((bm, bn), lambda i, j: (i, j)),
        ],
        out_specs=pl.BlockSpec((bm, 128), lambda i, j: (i, 0)),
        out_shape=jax.ShapeDtypeStruct((m, 128), jnp.int32),
        scratch_shapes=[
            pltpu.VMEM((bm, 1), jnp.float32),
            pltpu.VMEM((bm, 1), jnp.int32),
        ],
    )(xf, protos_t, g)


_SC_CHUNK = 32


def _gather_body(b_per_w, d, table_hbm, idx_hbm, out_hbm, idx_v, rows_v, sem):
    wid = lax.axis_index("s") * 2 + lax.axis_index("c")
    base = wid * b_per_w
    pltpu.sync_copy(idx_hbm.at[pl.ds(base, b_per_w)], idx_v)
    for c in range(b_per_w // _SC_CHUNK):
        pltpu.async_copy(
            table_hbm.at[idx_v.at[pl.ds(c * _SC_CHUNK, _SC_CHUNK)]],
            rows_v, sem).wait()
        pltpu.sync_copy(rows_v,
                        out_hbm.at[pl.ds(base + c * _SC_CHUNK, _SC_CHUNK)])


def _gather_call(prototypes, idx):
    b = idx.shape[0]
    d = prototypes.shape[1]
    nw = 32
    b_per_w = b // nw
    mesh = plsc.VectorSubcoreMesh(core_axis_name="c", subcore_axis_name="s")
    body = functools.partial(_gather_body, b_per_w, d)
    return pl.kernel(
        body,
        out_type=jax.ShapeDtypeStruct((b, d), jnp.float32),
        mesh=mesh,
        scratch_types=[
            pltpu.VMEM((b_per_w,), jnp.int32),
            pltpu.VMEM((_SC_CHUNK, d), jnp.float32),
            pltpu.SemaphoreType.DMA,
        ],
    )(prototypes, idx)


def _residual_body(x_ref, p_ref, o_ref, h_ref):
    xr = x_ref[...] - p_ref[...]
    o_ref[...] = xr
    d2 = jnp.sum(xr * xr, axis=1, keepdims=True)
    dist = jnp.sqrt(jnp.maximum(d2, 0.0)) + 1e-20
    hsel = (1.0 / 1.0) * (jnp.exp(1.0 * dist) + jnp.exp(-1.0 * dist) - (-1.0))
    h_ref[...] = jnp.broadcast_to(hsel, h_ref.shape)


def _residual_call(xf, proto, bm):
    m, k = xf.shape
    return pl.pallas_call(
        _residual_body,
        grid=(m // bm,),
        in_specs=[
            pl.BlockSpec((bm, k), lambda i: (i, 0)),
            pl.BlockSpec((bm, k), lambda i: (i, 0)),
        ],
        out_specs=[
            pl.BlockSpec((bm, k), lambda i: (i, 0)),
            pl.BlockSpec((bm, 128), lambda i: (i, 0)),
        ],
        out_shape=[
            jax.ShapeDtypeStruct((m, k), jnp.float32),
            jax.ShapeDtypeStruct((m, 128), jnp.float32),
        ],
    )(xf, proto)


def kernel(x, prototypes):
    batch, seq, hidden = x.shape
    m = batch * seq
    xf = x.reshape(m, hidden)
    pt = prototypes.T

    idx_w, vmax_w = _select_call(xf, pt, bm=1024, bn=1024)
    idx_fast = idx_w[:, 0]
    vmax = vmax_w[:, 0]

    def _with_gumbel():
        u = jax.random.uniform(jax.random.key(42), (m, prototypes.shape[0]),
                               minval=1e-20, maxval=1.0)
        g = -jnp.log(-jnp.log(u))
        return _select_g_call(xf, pt, g, bm=1024, bn=1024)[:, 0]

    need_gumbel = jnp.any(vmax < jnp.float32(_VMAX_SAFE))
    idx = lax.cond(need_gumbel, _with_gumbel, lambda: idx_fast)

    proto = _gather_call(prototypes, idx)
    xr, hsel_w = _residual_call(xf, proto, bm=512)
    hsel = hsel_w[:, :1]

    return (proto.reshape(batch, seq, hidden),
            xr.reshape(batch, seq, hidden),
            hsel)
